# baseline (speedup 1.0000x reference)
import functools

import jax
import jax.numpy as jnp
from jax import lax
from jax.experimental import pallas as pl
from jax.experimental.pallas import tpu as pltpu
from jax.experimental.pallas import tpu_sc as plsc

_TILE_V = 2048
_NBUF = 6


def _sc_gather(table, idx):
    B = idx.shape[0]
    V, D = table.shape
    info = plsc.get_sparse_core_info()
    nc, ns = info.num_cores, info.num_subcores
    nw = nc * ns
    b_per_w = B // nw
    mesh = plsc.VectorSubcoreMesh(core_axis_name="c", subcore_axis_name="s")

    @functools.partial(
        pl.kernel,
        mesh=mesh,
        compiler_params=pltpu.CompilerParams(use_tc_tiling_on_sc=False),
        out_type=jax.ShapeDtypeStruct((B, D), jnp.float32),
        scratch_types=[
            pltpu.VMEM((b_per_w,), jnp.int32),
            pltpu.VMEM((b_per_w, D), jnp.float32),
            pltpu.SemaphoreType.DMA,
        ],
    )
    def gather_kernel(table_hbm, idx_hbm, out_hbm, idx_v, rows_v, sem):
        wid = lax.axis_index("s") * nc + lax.axis_index("c")
        base = wid * b_per_w
        pltpu.sync_copy(idx_hbm.at[pl.ds(base, b_per_w)], idx_v)
        pltpu.async_copy(table_hbm.at[idx_v], rows_v, sem).wait()
        pltpu.sync_copy(rows_v, out_hbm.at[pl.ds(base, b_per_w)])

    return gather_kernel(table, idx)


def _make_mm_body(n_tiles, tile_v, tail_v, nbuf, B):
    def body(e_ref, wt_ref, b_ref, o_hbm, acc_ref, sems):
        j = pl.program_id(0)
        slot = lax.rem(j, nbuf)

        @pl.when(j >= nbuf)
        def _():
            pltpu.make_async_copy(
                acc_ref.at[slot], o_hbm.at[pl.ds(0, tile_v)], sems.at[slot]
            ).wait()

        acc = lax.dot_general(
            wt_ref[...], e_ref[...],
            dimension_numbers=(((0,), (1,)), ((), ())),
            preferred_element_type=jnp.float32,
        )
        acc = acc + lax.dot_general(
            b_ref[...], jnp.ones((1, B), jnp.float32),
            dimension_numbers=(((0,), (0,)), ((), ())),
            preferred_element_type=jnp.float32,
        )
        acc_ref[slot] = acc

        @pl.when(j < n_tiles - 1)
        def _():
            pltpu.make_async_copy(
                acc_ref.at[slot],
                o_hbm.at[pl.ds(j * tile_v, tile_v)],
                sems.at[slot],
            ).start()

        @pl.when(j == n_tiles - 1)
        def _():
            pltpu.make_async_copy(
                acc_ref.at[slot, pl.ds(0, tail_v)],
                o_hbm.at[pl.ds(j * tile_v, tail_v)],
                sems.at[slot],
            ).start()
            for step in range(max(n_tiles - nbuf, 0), n_tiles - 1):
                s = step % nbuf
                pltpu.make_async_copy(
                    acc_ref.at[s], o_hbm.at[pl.ds(0, tile_v)], sems.at[s]
                ).wait()
            pltpu.make_async_copy(
                acc_ref.at[slot, pl.ds(0, tail_v)],
                o_hbm.at[pl.ds(0, tail_v)],
                sems.at[slot],
            ).wait()

    return body


def _projection(embeds, linear_w, linear_b, tile_v=_TILE_V, nbuf=_NBUF):
    B, D = embeds.shape
    V = linear_w.shape[0]
    n_tiles = pl.cdiv(V, tile_v)
    tail_v = V - (n_tiles - 1) * tile_v

    wt = linear_w.T
    bias2d = linear_b.reshape(1, V)

    out_t = pl.pallas_call(
        _make_mm_body(n_tiles, tile_v, tail_v, nbuf, B),
        grid=(n_tiles,),
        in_specs=[
            pl.BlockSpec((B, D), lambda j: (0, 0)),
            pl.BlockSpec((D, tile_v), lambda j: (0, j)),
            pl.BlockSpec((1, tile_v), lambda j: (0, j)),
        ],
        out_specs=pl.BlockSpec(memory_space=pl.ANY),
        out_shape=jax.ShapeDtypeStruct((V, B), jnp.float32),
        scratch_shapes=[
            pltpu.VMEM((nbuf, tile_v, B), jnp.float32),
            pltpu.SemaphoreType.DMA((nbuf,)),
        ],
    )(embeds, wt, bias2d)
    return out_t.T


def kernel(inputs, embedding_table, linear_w, linear_b):
    idx = inputs.astype(jnp.int32)
    embeds = _sc_gather(embedding_table, idx)
    return _projection(embeds, linear_w, linear_b)


# speedup vs baseline: 1.0093x; 1.0093x over previous
import functools

import jax
import jax.numpy as jnp
from jax import lax
from jax.experimental import pallas as pl
from jax.experimental.pallas import tpu as pltpu
from jax.experimental.pallas import tpu_sc as plsc

_TILE_V = 2048
_NBUF = 4
_TILE_U = 2048
_NBUF_U = 4


def _make_untile_body(n_tiles, tile_u, tail_u, nbuf, D):
    def body(tt_ref, o_hbm, buf_ref, sems):
        j = pl.program_id(0)
        slot = lax.rem(j, nbuf)

        @pl.when(j >= nbuf)
        def _():
            pltpu.make_async_copy(
                buf_ref.at[slot], o_hbm.at[pl.ds(0, tile_u // 2)], sems.at[slot]
            ).wait()

        buf_ref[slot] = tt_ref[...].T.reshape(tile_u // 2, 2 * D)

        @pl.when(j < n_tiles - 1)
        def _():
            pltpu.make_async_copy(
                buf_ref.at[slot],
                o_hbm.at[pl.ds(j * (tile_u // 2), tile_u // 2)],
                sems.at[slot],
            ).start()

        @pl.when(j == n_tiles - 1)
        def _():
            pltpu.make_async_copy(
                buf_ref.at[slot, pl.ds(0, tail_u // 2)],
                o_hbm.at[pl.ds(j * (tile_u // 2), tail_u // 2)],
                sems.at[slot],
            ).start()
            for step in range(max(n_tiles - nbuf, 0), n_tiles - 1):
                s = step % nbuf
                pltpu.make_async_copy(
                    buf_ref.at[s], o_hbm.at[pl.ds(0, tile_u // 2)], sems.at[s]
                ).wait()
            pltpu.make_async_copy(
                buf_ref.at[slot, pl.ds(0, tail_u // 2)],
                o_hbm.at[pl.ds(0, tail_u // 2)],
                sems.at[slot],
            ).wait()

    return body


def _untile_table(table, tile_u=_TILE_U, nbuf=_NBUF_U):
    V, D = table.shape
    tt = table.T
    n_tiles = pl.cdiv(V, tile_u)
    tail_u = V - (n_tiles - 1) * tile_u
    return pl.pallas_call(
        _make_untile_body(n_tiles, tile_u, tail_u, nbuf, D),
        grid=(n_tiles,),
        in_specs=[pl.BlockSpec((D, tile_u), lambda j: (0, j))],
        out_specs=pl.BlockSpec(memory_space=pl.ANY),
        out_shape=jax.ShapeDtypeStruct((V // 2, 2 * D), jnp.float32),
        scratch_shapes=[
            pltpu.VMEM((nbuf, tile_u // 2, 2 * D), jnp.float32),
            pltpu.SemaphoreType.DMA((nbuf,)),
        ],
    )(tt)


def _sc_gather_wide(table2, idx2):
    B = idx2.shape[0]
    V2, D2 = table2.shape
    info = plsc.get_sparse_core_info()
    nc, ns = info.num_cores, info.num_subcores
    nw = nc * ns
    b_per_w = B // nw
    mesh = plsc.VectorSubcoreMesh(core_axis_name="c", subcore_axis_name="s")

    @functools.partial(
        pl.kernel,
        mesh=mesh,
        compiler_params=pltpu.CompilerParams(use_tc_tiling_on_sc=True),
        out_type=jax.ShapeDtypeStruct((B, D2), jnp.float32),
        scratch_types=[
            pltpu.VMEM((b_per_w,), jnp.int32),
            pltpu.VMEM((b_per_w, D2), jnp.float32),
            pltpu.SemaphoreType.DMA,
        ],
    )
    def gather_kernel(table_hbm, idx_hbm, out_hbm, idx_v, rows_v, sem):
        wid = lax.axis_index("s") * nc + lax.axis_index("c")
        base = wid * b_per_w
        pltpu.sync_copy(idx_hbm.at[pl.ds(base, b_per_w)], idx_v)
        pltpu.async_copy(table_hbm.at[idx_v], rows_v, sem).wait()
        pltpu.sync_copy(rows_v, out_hbm.at[pl.ds(base, b_per_w)])

    return gather_kernel(table2, idx2)


def _make_mm_body(n_tiles, tile_v, tail_v, nbuf, B, D):
    def body(wide_ref, par_ref, wt_ref, b_ref, o_hbm, e_ref, acc_ref, sems):
        j = pl.program_id(0)
        slot = lax.rem(j, nbuf)

        @pl.when(j == 0)
        def _():
            wide = wide_ref[...]
            e_ref[...] = jnp.where(
                par_ref[...] > 0, wide[:, D:2 * D], wide[:, 0:D]
            )

        @pl.when(j >= nbuf)
        def _():
            pltpu.make_async_copy(
                acc_ref.at[slot], o_hbm.at[pl.ds(0, tile_v)], sems.at[slot]
            ).wait()

        acc = lax.dot_general(
            wt_ref[...], e_ref[...],
            dimension_numbers=(((0,), (1,)), ((), ())),
            preferred_element_type=jnp.float32,
        )
        acc = acc + lax.dot_general(
            b_ref[...], jnp.ones((1, B), jnp.float32),
            dimension_numbers=(((0,), (0,)), ((), ())),
            preferred_element_type=jnp.float32,
        )
        acc_ref[slot] = acc

        @pl.when(j < n_tiles - 1)
        def _():
            pltpu.make_async_copy(
                acc_ref.at[slot],
                o_hbm.at[pl.ds(j * tile_v, tile_v)],
                sems.at[slot],
            ).start()

        @pl.when(j == n_tiles - 1)
        def _():
            pltpu.make_async_copy(
                acc_ref.at[slot, pl.ds(0, tail_v)],
                o_hbm.at[pl.ds(j * tile_v, tail_v)],
                sems.at[slot],
            ).start()
            for step in range(max(n_tiles - nbuf, 0), n_tiles - 1):
                s = step % nbuf
                pltpu.make_async_copy(
                    acc_ref.at[s], o_hbm.at[pl.ds(0, tile_v)], sems.at[s]
                ).wait()
            pltpu.make_async_copy(
                acc_ref.at[slot, pl.ds(0, tail_v)],
                o_hbm.at[pl.ds(0, tail_v)],
                sems.at[slot],
            ).wait()

    return body


def _projection(wide, par, linear_w, linear_b, tile_v=_TILE_V, nbuf=_NBUF):
    B, D2 = wide.shape
    D = D2 // 2
    V = linear_w.shape[0]
    n_tiles = pl.cdiv(V, tile_v)
    tail_v = V - (n_tiles - 1) * tile_v

    wt = linear_w.T
    bias2d = linear_b.reshape(1, V)

    out_t = pl.pallas_call(
        _make_mm_body(n_tiles, tile_v, tail_v, nbuf, B, D),
        grid=(n_tiles,),
        in_specs=[
            pl.BlockSpec((B, D2), lambda j: (0, 0)),
            pl.BlockSpec((B, 1), lambda j: (0, 0)),
            pl.BlockSpec((D, tile_v), lambda j: (0, j)),
            pl.BlockSpec((1, tile_v), lambda j: (0, j)),
        ],
        out_specs=pl.BlockSpec(memory_space=pl.ANY),
        out_shape=jax.ShapeDtypeStruct((V, B), jnp.float32),
        scratch_shapes=[
            pltpu.VMEM((B, D), jnp.float32),
            pltpu.VMEM((nbuf, tile_v, B), jnp.float32),
            pltpu.SemaphoreType.DMA((nbuf,)),
        ],
    )(wide, par, wt, bias2d)
    return out_t.T


def kernel(inputs, embedding_table, linear_w, linear_b):
    idx = inputs.astype(jnp.int32)
    V, D = embedding_table.shape
    table2 = embedding_table.reshape(V // 2, 2 * D)
    wide = _sc_gather_wide(table2, idx // 2)
    par = (idx % 2).astype(jnp.int32).reshape(idx.shape[0], 1)
    return _projection(wide, par, linear_w, linear_b)


# speedup vs baseline: 1.1051x; 1.0949x over previous
import functools

import jax
import jax.numpy as jnp
from jax import lax
from jax.experimental import pallas as pl
from jax.experimental.pallas import tpu as pltpu
from jax.experimental.pallas import tpu_sc as plsc

_TILE_V = 2048
_NBUF = 4
_TILE_U = 2048
_NBUF_U = 4


def _make_untile_body(n_tiles, tile_u, tail_u, nbuf, D):
    def body(tt_ref, o_hbm, buf_ref, sems):
        j = pl.program_id(0)
        slot = lax.rem(j, nbuf)

        @pl.when(j >= nbuf)
        def _():
            pltpu.make_async_copy(
                buf_ref.at[slot], o_hbm.at[pl.ds(0, tile_u)], sems.at[slot]
            ).wait()

        buf_ref[slot, :, 0:D] = tt_ref[...].T

        @pl.when(j < n_tiles - 1)
        def _():
            pltpu.make_async_copy(
                buf_ref.at[slot],
                o_hbm.at[pl.ds(j * tile_u, tile_u)],
                sems.at[slot],
            ).start()

        @pl.when(j == n_tiles - 1)
        def _():
            pltpu.make_async_copy(
                buf_ref.at[slot, pl.ds(0, tail_u)],
                o_hbm.at[pl.ds(j * tile_u, tail_u)],
                sems.at[slot],
            ).start()
            for step in range(max(n_tiles - nbuf, 0), n_tiles - 1):
                s = step % nbuf
                pltpu.make_async_copy(
                    buf_ref.at[s], o_hbm.at[pl.ds(0, tile_u)], sems.at[s]
                ).wait()
            pltpu.make_async_copy(
                buf_ref.at[slot, pl.ds(0, tail_u)],
                o_hbm.at[pl.ds(0, tail_u)],
                sems.at[slot],
            ).wait()

    return body


def _untile_table(table, tile_u=_TILE_U, nbuf=_NBUF_U):
    V, D = table.shape
    tt = table.T
    n_tiles = pl.cdiv(V, tile_u)
    tail_u = V - (n_tiles - 1) * tile_u
    return pl.pallas_call(
        _make_untile_body(n_tiles, tile_u, tail_u, nbuf, D),
        grid=(n_tiles,),
        in_specs=[pl.BlockSpec((D, tile_u), lambda j: (0, j))],
        out_specs=pl.BlockSpec(memory_space=pl.ANY),
        out_shape=jax.ShapeDtypeStruct((V, 128), jnp.float32),
        scratch_shapes=[
            pltpu.VMEM((nbuf, tile_u, 128), jnp.float32),
            pltpu.SemaphoreType.DMA((nbuf,)),
        ],
    )(tt)


def _sc_gather_wide(tablew, idx):
    B = idx.shape[0]
    Vw, Dw = tablew.shape
    info = plsc.get_sparse_core_info()
    nc, ns = info.num_cores, info.num_subcores
    nw = nc * ns
    b_per_w = B // nw
    mesh = plsc.VectorSubcoreMesh(core_axis_name="c", subcore_axis_name="s")

    @functools.partial(
        pl.kernel,
        mesh=mesh,
        compiler_params=pltpu.CompilerParams(use_tc_tiling_on_sc=True),
        out_type=jax.ShapeDtypeStruct((B, Dw), jnp.float32),
        scratch_types=[
            pltpu.VMEM((b_per_w,), jnp.int32),
            pltpu.VMEM((b_per_w, Dw), jnp.float32),
            pltpu.SemaphoreType.DMA,
        ],
    )
    def gather_kernel(table_hbm, idx_hbm, out_hbm, idx_v, rows_v, sem):
        wid = lax.axis_index("s") * nc + lax.axis_index("c")
        base = wid * b_per_w
        pltpu.sync_copy(idx_hbm.at[pl.ds(base, b_per_w)], idx_v)
        pltpu.async_copy(table_hbm.at[idx_v], rows_v, sem).wait()
        pltpu.sync_copy(rows_v, out_hbm.at[pl.ds(base, b_per_w)])

    return gather_kernel(tablew, idx)


def _make_mm_body(n_tiles, tile_v, tail_v, nbuf, B, D):
    def body(wide_ref, wt_ref, b_ref, o_hbm, e_ref, acc_ref, sems):
        j = pl.program_id(0)
        slot = lax.rem(j, nbuf)

        @pl.when(j == 0)
        def _():
            e_ref[...] = wide_ref[:, 0:D]

        @pl.when(j >= nbuf)
        def _():
            pltpu.make_async_copy(
                acc_ref.at[slot], o_hbm.at[pl.ds(0, tile_v)], sems.at[slot]
            ).wait()

        acc = lax.dot_general(
            wt_ref[...], e_ref[...],
            dimension_numbers=(((0,), (1,)), ((), ())),
            preferred_element_type=jnp.float32,
        )
        acc = acc + lax.dot_general(
            b_ref[...], jnp.ones((1, B), jnp.float32),
            dimension_numbers=(((0,), (0,)), ((), ())),
            preferred_element_type=jnp.float32,
        )
        acc_ref[slot] = acc

        @pl.when(j < n_tiles - 1)
        def _():
            pltpu.make_async_copy(
                acc_ref.at[slot],
                o_hbm.at[pl.ds(j * tile_v, tile_v)],
                sems.at[slot],
            ).start()

        @pl.when(j == n_tiles - 1)
        def _():
            pltpu.make_async_copy(
                acc_ref.at[slot, pl.ds(0, tail_v)],
                o_hbm.at[pl.ds(j * tile_v, tail_v)],
                sems.at[slot],
            ).start()
            for step in range(max(n_tiles - nbuf, 0), n_tiles - 1):
                s = step % nbuf
                pltpu.make_async_copy(
                    acc_ref.at[s], o_hbm.at[pl.ds(0, tile_v)], sems.at[s]
                ).wait()
            pltpu.make_async_copy(
                acc_ref.at[slot, pl.ds(0, tail_v)],
                o_hbm.at[pl.ds(0, tail_v)],
                sems.at[slot],
            ).wait()

    return body


def _projection(wide, linear_w, linear_b, tile_v=_TILE_V, nbuf=_NBUF):
    B, Dw = wide.shape
    V, D = linear_w.shape
    n_tiles = pl.cdiv(V, tile_v)
    tail_v = V - (n_tiles - 1) * tile_v

    wt = linear_w.T
    bias2d = linear_b.reshape(1, V)

    out_t = pl.pallas_call(
        _make_mm_body(n_tiles, tile_v, tail_v, nbuf, B, D),
        grid=(n_tiles,),
        in_specs=[
            pl.BlockSpec((B, Dw), lambda j: (0, 0)),
            pl.BlockSpec((D, tile_v), lambda j: (0, j)),
            pl.BlockSpec((1, tile_v), lambda j: (0, j)),
        ],
        out_specs=pl.BlockSpec(memory_space=pl.ANY),
        out_shape=jax.ShapeDtypeStruct((V, B), jnp.float32),
        scratch_shapes=[
            pltpu.VMEM((B, D), jnp.float32),
            pltpu.VMEM((nbuf, tile_v, B), jnp.float32),
            pltpu.SemaphoreType.DMA((nbuf,)),
        ],
    )(wide, wt, bias2d)
    return out_t.T


def kernel(inputs, embedding_table, linear_w, linear_b):
    idx = inputs.astype(jnp.int32)
    tablew = _untile_table(embedding_table)
    wide = _sc_gather_wide(tablew, idx)
    return _projection(wide, linear_w, linear_b)


# speedup vs baseline: 1.1713x; 1.0599x over previous
import functools

import jax
import jax.numpy as jnp
from jax import lax
from jax.experimental import pallas as pl
from jax.experimental.pallas import tpu as pltpu
from jax.experimental.pallas import tpu_sc as plsc

_TILE_V = 2048
_NBUF = 4
_TILE_U = 2048
_NBUF_U = 4


def _make_untile_body(n_tiles, tile_u, tail_u, nbuf, D):
    def body(tta_ref, ttb_ref, o_hbm, buf_ref, sems):
        j = pl.program_id(0)
        slot = lax.rem(j, nbuf)

        @pl.when(j >= nbuf)
        def _():
            pltpu.make_async_copy(
                buf_ref.at[slot], o_hbm.at[pl.ds(0, 2 * tile_u)], sems.at[slot]
            ).wait()

        buf_ref[slot, 0:tile_u, 0:D] = tta_ref[...].T
        buf_ref[slot, tile_u:2 * tile_u, 0:D] = ttb_ref[...].T

        @pl.when(j < n_tiles - 1)
        def _():
            pltpu.make_async_copy(
                buf_ref.at[slot],
                o_hbm.at[pl.ds(j * (2 * tile_u), 2 * tile_u)],
                sems.at[slot],
            ).start()

        @pl.when(j == n_tiles - 1)
        def _():
            pltpu.make_async_copy(
                buf_ref.at[slot, pl.ds(0, tail_u)],
                o_hbm.at[pl.ds(j * (2 * tile_u), tail_u)],
                sems.at[slot],
            ).start()
            for step in range(max(n_tiles - nbuf, 0), n_tiles - 1):
                s = step % nbuf
                pltpu.make_async_copy(
                    buf_ref.at[s], o_hbm.at[pl.ds(0, 2 * tile_u)], sems.at[s]
                ).wait()
            pltpu.make_async_copy(
                buf_ref.at[slot, pl.ds(0, tail_u)],
                o_hbm.at[pl.ds(0, tail_u)],
                sems.at[slot],
            ).wait()

    return body


def _untile_table(table, tile_u=_TILE_U, nbuf=_NBUF_U):
    V, D = table.shape
    tt = table.T
    n_tiles = pl.cdiv(V, 2 * tile_u)
    tail_u = V - (n_tiles - 1) * 2 * tile_u
    max_blk = pl.cdiv(V, tile_u) - 1
    return pl.pallas_call(
        _make_untile_body(n_tiles, tile_u, tail_u, nbuf, D),
        grid=(n_tiles,),
        in_specs=[
            pl.BlockSpec(
                (D, tile_u),
                lambda j: (0, jnp.minimum(2 * j, max_blk))),
            pl.BlockSpec(
                (D, tile_u),
                lambda j: (0, jnp.minimum(2 * j + 1, max_blk))),
        ],
        out_specs=pl.BlockSpec(memory_space=pl.ANY),
        out_shape=jax.ShapeDtypeStruct((V, 128), jnp.float32),
        scratch_shapes=[
            pltpu.VMEM((nbuf, 2 * tile_u, 128), jnp.float32),
            pltpu.SemaphoreType.DMA((nbuf,)),
        ],
    )(tt, tt)


def _sc_gather_wide(tablew, idx):
    B = idx.shape[0]
    Vw, Dw = tablew.shape
    info = plsc.get_sparse_core_info()
    nc, ns = info.num_cores, info.num_subcores
    nw = nc * ns
    b_per_w = B // nw
    mesh = plsc.VectorSubcoreMesh(core_axis_name="c", subcore_axis_name="s")

    @functools.partial(
        pl.kernel,
        mesh=mesh,
        compiler_params=pltpu.CompilerParams(use_tc_tiling_on_sc=True),
        out_type=jax.ShapeDtypeStruct((B, Dw), jnp.float32),
        scratch_types=[
            pltpu.VMEM((b_per_w,), jnp.int32),
            pltpu.VMEM((b_per_w, Dw), jnp.float32),
            pltpu.SemaphoreType.DMA,
        ],
    )
    def gather_kernel(table_hbm, idx_hbm, out_hbm, idx_v, rows_v, sem):
        wid = lax.axis_index("s") * nc + lax.axis_index("c")
        base = wid * b_per_w
        pltpu.sync_copy(idx_hbm.at[pl.ds(base, b_per_w)], idx_v)
        pltpu.async_copy(table_hbm.at[idx_v], rows_v, sem).wait()
        pltpu.sync_copy(rows_v, out_hbm.at[pl.ds(base, b_per_w)])

    return gather_kernel(tablew, idx)


def _make_mm_body(n_tiles, tile_v, tail_v, nbuf, B, D):
    def body(wide_ref, wt_ref, b_ref, o_hbm, e_ref, acc_ref, sems):
        j = pl.program_id(0)
        slot = lax.rem(j, nbuf)

        @pl.when(j == 0)
        def _():
            e_ref[...] = wide_ref[:, 0:D]

        @pl.when(j >= nbuf)
        def _():
            pltpu.make_async_copy(
                acc_ref.at[slot], o_hbm.at[pl.ds(0, tile_v)], sems.at[slot]
            ).wait()

        acc = lax.dot_general(
            wt_ref[...], e_ref[...],
            dimension_numbers=(((0,), (1,)), ((), ())),
            preferred_element_type=jnp.float32,
        )
        acc = acc + lax.dot_general(
            b_ref[...], jnp.ones((1, B), jnp.float32),
            dimension_numbers=(((0,), (0,)), ((), ())),
            preferred_element_type=jnp.float32,
        )
        acc_ref[slot] = acc

        @pl.when(j < n_tiles - 1)
        def _():
            pltpu.make_async_copy(
                acc_ref.at[slot],
                o_hbm.at[pl.ds(j * tile_v, tile_v)],
                sems.at[slot],
            ).start()

        @pl.when(j == n_tiles - 1)
        def _():
            pltpu.make_async_copy(
                acc_ref.at[slot, pl.ds(0, tail_v)],
                o_hbm.at[pl.ds(j * tile_v, tail_v)],
                sems.at[slot],
            ).start()
            for step in range(max(n_tiles - nbuf, 0), n_tiles - 1):
                s = step % nbuf
                pltpu.make_async_copy(
                    acc_ref.at[s], o_hbm.at[pl.ds(0, tile_v)], sems.at[s]
                ).wait()
            pltpu.make_async_copy(
                acc_ref.at[slot, pl.ds(0, tail_v)],
                o_hbm.at[pl.ds(0, tail_v)],
                sems.at[slot],
            ).wait()

    return body


def _projection(wide, linear_w, linear_b, tile_v=_TILE_V, nbuf=_NBUF):
    B, Dw = wide.shape
    V, D = linear_w.shape
    n_tiles = pl.cdiv(V, tile_v)
    tail_v = V - (n_tiles - 1) * tile_v

    wt = linear_w.T
    bias2d = linear_b.reshape(1, V)

    out_t = pl.pallas_call(
        _make_mm_body(n_tiles, tile_v, tail_v, nbuf, B, D),
        grid=(n_tiles,),
        in_specs=[
            pl.BlockSpec((B, Dw), lambda j: (0, 0)),
            pl.BlockSpec((D, tile_v), lambda j: (0, j)),
            pl.BlockSpec((1, tile_v), lambda j: (0, j)),
        ],
        out_specs=pl.BlockSpec(memory_space=pl.ANY),
        out_shape=jax.ShapeDtypeStruct((V, B), jnp.float32),
        scratch_shapes=[
            pltpu.VMEM((B, D), jnp.float32),
            pltpu.VMEM((nbuf, tile_v, B), jnp.float32),
            pltpu.SemaphoreType.DMA((nbuf,)),
        ],
    )(wide, wt, bias2d)
    return out_t.T


def kernel(inputs, embedding_table, linear_w, linear_b):
    idx = inputs.astype(jnp.int32)
    tablew = _untile_table(embedding_table)
    wide = _sc_gather_wide(tablew, idx)
    return _projection(wide, linear_w, linear_b)
